# baseline (speedup 1.0000x reference)
import jax
import jax.numpy as jnp
from jax.experimental import pallas as pl
from jax.experimental.pallas import tpu as pltpu

_K = 64
_ROWS_PER_BLOCK = 64
_LANES = 128
_TOP_PER_LANE = 4
_IMIN = jnp.iinfo(jnp.int32).min


def _to_key(v):
    i = jax.lax.bitcast_convert_type(v, jnp.int32)
    return jnp.where(i >= 0, i, jnp.bitwise_xor(jnp.bitwise_not(i), jnp.int32(_IMIN)))


def _from_key(k):
    i = jnp.where(k >= 0, k, jnp.bitwise_not(jnp.bitwise_xor(k, jnp.int32(_IMIN))))
    return jax.lax.bitcast_convert_type(i, jnp.float32)


def _topk_mask_body(x_ref, o_ref):
    x = x_ref[...]
    nrows, ncols = x.shape
    nchunks = ncols // _LANES
    neg_inf = jnp.float32(-jnp.inf)

    chunks = [x[:, c * _LANES:(c + 1) * _LANES] for c in range(nchunks)]

    def tree_max(vals):
        while len(vals) > 1:
            nxt = [jnp.maximum(vals[j], vals[j + 1]) for j in range(0, len(vals) - 1, 2)]
            if len(vals) % 2:
                nxt.append(vals[-1])
            vals = nxt
        return vals[0]

    nf = nchunks // 4
    fams = [chunks[i * nf:(i + 1) * nf] for i in range(4)]
    tops = []
    prevs = [None, None, None, None]
    for _ in range(_TOP_PER_LANE):
        for f in range(4):
            if prevs[f] is None:
                cur = tree_max(fams[f])
            else:
                cur = tree_max([jnp.where(c < prevs[f], c, neg_inf) for c in fams[f]])
            prevs[f] = cur
            tops.append(cur)
    cand = _to_key(jnp.concatenate(tops, axis=1))
    kmax = jnp.max(cand, axis=1, keepdims=True)

    def small_cond(carry):
        glo, ghi, gfound, gthr = carry
        return jnp.any((gfound == 0) & ((ghi - 1) > glo))

    def small_body(carry):
        glo, ghi, gfound, gthr = carry
        gmid = (glo >> 1) + (ghi >> 1) + (glo & ghi & 1)
        gcnt = jnp.sum(jnp.where(cand >= gmid, 1.0, 0.0), axis=1, keepdims=True)
        hit = gcnt == float(_K)
        ge = gcnt >= float(_K)
        gthr = jnp.where(hit & (gfound == 0), gmid, gthr)
        gfound = gfound | hit.astype(jnp.int32)
        glo = jnp.where(ge, gmid, glo)
        ghi = jnp.where(ge, ghi, gmid)
        return glo, ghi, gfound, gthr

    cmin = jnp.min(cand, axis=1, keepdims=True)
    glo, _, gfound, gthr = jax.lax.while_loop(
        small_cond, small_body,
        (cmin, kmax + 1, jnp.zeros((nrows, 1), jnp.int32), cmin))
    tstar = jnp.where(gfound == 1, gthr, glo)

    def cnt_ge(t_key):
        t = _from_key(t_key)
        return jnp.sum(jnp.where(x >= t, 1.0, 0.0), axis=1, keepdims=True)

    def cond(carry):
        lo, hi, found, thr, cnt_lo = carry
        return jnp.any((found == 0) & ((hi - 1) > lo))

    def body(carry):
        lo, hi, found, thr, cnt_lo = carry
        mid = (lo >> 1) + (hi >> 1) + (lo & hi & 1)
        cnt = cnt_ge(mid)
        hit = cnt == float(_K)
        ge = cnt >= float(_K)
        thr = jnp.where(hit & (found == 0), mid, thr)
        found = found | hit.astype(jnp.int32)
        lo = jnp.where(ge, mid, lo)
        hi = jnp.where(ge, hi, mid)
        cnt_lo = jnp.where(ge, cnt, cnt_lo)
        return lo, hi, found, thr, cnt_lo

    cnt0 = cnt_ge(tstar)
    found0 = (cnt0 == float(_K)).astype(jnp.int32)
    carry0 = (tstar, kmax + 1, found0, tstar, cnt0)
    lo, hi, found, thr, cnt_lo = jax.lax.while_loop(cond, body, carry0)
    is_found = found == 1
    thr_f = _from_key(jnp.where(is_found, thr, lo))
    n_ge = jnp.where(is_found, float(_K), cnt_lo)
    overflow = jnp.any(n_ge > float(_K))

    @pl.when(jnp.logical_not(overflow))
    def _():
        o_ref[...] = jnp.where(x >= thr_f, jnp.maximum(x, 0.0), 0.0)

    @pl.when(overflow)
    def _():
        col = jax.lax.broadcasted_iota(jnp.int32, x.shape, 1)
        extra = n_ge.astype(jnp.int32) - _K
        tcol = jnp.where(x == thr_f, col, -1)
        cut = jnp.full((nrows, 1), jnp.iinfo(jnp.int32).max, jnp.int32)
        for _ in range(4):
            hi_col = jnp.max(jnp.where(tcol < cut, tcol, -1), axis=1, keepdims=True)
            cut = jnp.where(extra > 0, hi_col, cut)
            extra = jnp.maximum(extra - 1, 0)
        keep = (x > thr_f) | ((x == thr_f) & (col < cut))
        o_ref[...] = jnp.where(keep, jnp.maximum(x, 0.0), 0.0)


def kernel(x):
    m, n = x.shape
    grid = (m // _ROWS_PER_BLOCK,)
    return pl.pallas_call(
        _topk_mask_body,
        grid=grid,
        in_specs=[pl.BlockSpec((_ROWS_PER_BLOCK, n), lambda r: (r, 0))],
        out_specs=pl.BlockSpec((_ROWS_PER_BLOCK, n), lambda r: (r, 0)),
        out_shape=jax.ShapeDtypeStruct((m, n), x.dtype),
        compiler_params=pltpu.CompilerParams(
            dimension_semantics=("parallel",),
        ),
    )(x)
